# Initial kernel scaffold; baseline (speedup 1.0000x reference)
import jax
import jax.numpy as jnp
from jax.experimental import pallas as pl


def kernel(x, edge_index, batch, W1, b1, W2, b2, W3, b3, fW1, fb1, fW2, fb2, fW3, fb3):
    raise NotImplementedError("write your pallas kernel here")


# speedup vs baseline: 10.6085x; 10.6085x over previous
import functools

import jax
import jax.numpy as jnp
from jax import lax
from jax.experimental import pallas as pl
from jax.experimental.pallas import tpu as pltpu
from jax.experimental.pallas import tpu_sc as plsc

N = 10000
E = 320000
G = 64
F = 128

NC = 2
NS = 16
NW = NC * NS
EPT = E // NW
EPS = E // NS
K = 80
NCH = EPS // K
HALF = 5120
PAD = 128
ACR = HALF + PAD
ZR = ACR // NS
WR = HALF // NS
AR = NC * HALF


@functools.cache
def _sc_mesh():
    return plsc.VectorSubcoreMesh(core_axis_name="c", subcore_axis_name="s",
                                  num_cores=NC, num_subcores=NS)


@functools.cache
def _make_sc_degree():
    @functools.partial(
        pl.kernel,
        out_type=jax.ShapeDtypeStruct((NW, 1, N), jnp.float32),
        mesh=_sc_mesh(),
        compiler_params=pltpu.CompilerParams(needs_layout_passes=False),
        scratch_types=[
            pltpu.VMEM((EPT,), jnp.int32),
            pltpu.VMEM((N,), jnp.float32),
        ],
    )
    def deg_kernel(dst_hbm, degp_hbm, dstv, deg):
        c = lax.axis_index("c")
        s = lax.axis_index("s")
        wid = s * NC + c

        def zero_body(i, _):
            deg[pl.ds(i * 16, 16)] = jnp.zeros((16,), jnp.float32)
            return 0

        lax.fori_loop(0, N // 16, zero_body, 0)
        pltpu.sync_copy(dst_hbm.at[pl.ds(wid * EPT, EPT)], dstv)
        ones = jnp.full((16,), 1.0, jnp.float32)

        def hist_body(i, _):
            iv = dstv[pl.ds(i * 16, 16)]
            plsc.addupdate_scatter(deg, [iv], ones)
            return 0

        lax.fori_loop(0, EPT // 16, hist_body, 0)
        pltpu.sync_copy(deg, degp_hbm.at[wid, 0])

    return deg_kernel


@functools.cache
def _make_sc_scatter():
    @functools.partial(
        pl.kernel,
        out_type=jax.ShapeDtypeStruct((AR, F), jnp.float32),
        mesh=_sc_mesh(),
        scratch_types=[
            pltpu.VMEM((2, 1, K), jnp.int32),
            pltpu.VMEM((2, 1, K), jnp.int32),
            pltpu.VMEM((2, K, F), jnp.float32),
            pltpu.VMEM((ZR, F), jnp.float32),
            pltpu.VMEM_SHARED((ACR, F), jnp.float32),
            pltpu.SemaphoreType.DMA,
            pltpu.SemaphoreType.DMA,
        ],
    )
    def scat(xs_hbm, src_hbm, dst_hbm, zrow_hbm, out_hbm,
             srcv, dstv, rows, bounce, acc, g0, g1):
        c = lax.axis_index("c")
        s = lax.axis_index("s")
        base = s * EPS
        row0 = c * HALF
        dump = HALF + s * 8
        gsem = (g0, g1)

        pltpu.sync_copy(zrow_hbm, bounce)
        pltpu.sync_copy(bounce, acc.at[pl.ds(s * ZR, ZR)])
        plsc.subcore_barrier()

        def load_and_gather(i, b):
            e0 = base + i * K
            pltpu.sync_copy(src_hbm.at[pl.ds(e0, K)], srcv.at[b, 0])
            pltpu.sync_copy(dst_hbm.at[pl.ds(e0, K)], dstv.at[b, 0])
            for j in range(K // 16):
                dv = dstv[b, 0, pl.ds(j * 16, 16)] - row0
                ok = (dv >= 0) & (dv < HALF)
                dstv[b, 0, pl.ds(j * 16, 16)] = jnp.where(ok, dv, dump)
            pltpu.async_copy(xs_hbm.at[srcv.at[b, 0]], rows.at[b], gsem[b])

        for b in range(2):
            load_and_gather(b, b)

        def consume(i, b):
            pltpu.make_async_copy(xs_hbm.at[srcv.at[b, 0]], rows.at[b],
                                  gsem[b]).wait()
            pltpu.sync_copy(rows.at[b], acc.at[dstv.at[b, 0]], add=True)

        def pair_body(t, _):
            for b in range(2):
                i = 2 * t + b
                consume(i, b)

                @pl.when(i + 2 < NCH)
                def _():
                    load_and_gather(i + 2, b)
            return 0

        lax.fori_loop(0, NCH // 2, pair_body, 0)
        if NCH % 2:
            consume(NCH - 1, 0)

        plsc.subcore_barrier()
        pltpu.sync_copy(acc.at[pl.ds(s * WR, WR)], bounce.at[pl.ds(0, WR)])
        pltpu.sync_copy(bounce.at[pl.ds(0, WR)],
                        out_hbm.at[pl.ds(row0 + s * WR, WR)])

    return scat


def _tc1_body(degp_ref, x_ref, dinv_ref, xs1_ref):
    ones = jnp.ones((NW, 1), jnp.float32)
    dp = degp_ref[...][:, 0, :]
    deg = lax.dot_general(dp, ones, (((0,), (0,)), ((), ())),
                          preferred_element_type=jnp.float32) + 1.0
    dinv = lax.rsqrt(deg)
    dinv_ref[...] = dinv
    xs1_ref[...] = x_ref[...] * dinv


def _tc_mid_body(s_ref, xs_ref, dinv_ref, wa_ref, ba_ref, *rest, pre_bias):
    dinv = dinv_ref[...]
    agg = (s_ref[...] + xs_ref[...]) * dinv
    if pre_bias:
        (out_ref,) = rest
        h = jnp.maximum(agg + ba_ref[...], 0.0)
        h = lax.dot_general(h, wa_ref[...], (((1,), (0,)), ((), ())),
                            preferred_element_type=jnp.float32)
        h = jnp.concatenate([h, jnp.zeros_like(h)], axis=1)
    else:
        wb_ref, out_ref = rest
        h = lax.dot_general(agg, wa_ref[...], (((1,), (0,)), ((), ())),
                            preferred_element_type=jnp.float32)
        h = jnp.maximum(h + ba_ref[...], 0.0)
        h = lax.dot_general(h, wb_ref[...], (((1,), (0,)), ((), ())),
                            preferred_element_type=jnp.float32)
    out_ref[...] = h * dinv


def _tc4_body(s_ref, xs_ref, dinv_ref, b3_ref, batch_ref,
              fw1_ref, fb1_ref, fw2_ref, fb2_ref, fw3_ref, fb3_ref, out_ref):
    dinv = dinv_ref[...]
    h3 = jnp.maximum((s_ref[...][:, :64] + xs_ref[...][:, :64]) * dinv
                     + b3_ref[...], 0.0)
    batch = batch_ref[...]

    gids = lax.broadcasted_iota(jnp.int32, (G, 1), 0)

    def seg_body(g, pool):
        mask = batch == g
        mg = jnp.max(jnp.where(mask, h3, -jnp.inf), axis=0)
        return jnp.where(gids == g, mg[None, :], pool)

    pool = lax.fori_loop(0, G, seg_body,
                         jnp.full((G, h3.shape[1]), -jnp.inf, jnp.float32))

    def mm(a, w_ref, b_ref):
        return lax.dot_general(a, w_ref[...], (((1,), (0,)), ((), ())),
                               preferred_element_type=jnp.float32) + b_ref[...]

    z = jnp.maximum(mm(pool, fw1_ref, fb1_ref), 0.0)
    z = jnp.maximum(mm(z, fw2_ref, fb2_ref), 0.0)
    z = mm(z, fw3_ref, fb3_ref)
    m = jnp.max(z, axis=1, keepdims=True)
    lse = m + jnp.log(jnp.sum(jnp.exp(z - m), axis=1, keepdims=True))
    out_ref[...] = z - lse


def _full(shape):
    return pl.BlockSpec(shape, lambda *_: tuple(0 for _ in shape))


def _tc1(degp, x):
    return pl.pallas_call(
        _tc1_body,
        out_shape=[jax.ShapeDtypeStruct((N, 1), jnp.float32),
                   jax.ShapeDtypeStruct((N, x.shape[1]), jnp.float32)],
    )(degp, x)


def _tc_mid(S, xs, dinv, wa, ba, wb, *, pre_bias, fout):
    B = 2000
    Fin = xs.shape[1]
    grid = (N // B,)
    in_specs = [
        pl.BlockSpec((B, F), lambda i: (i, 0)),
        pl.BlockSpec((B, Fin), lambda i: (i, 0)),
        pl.BlockSpec((B, 1), lambda i: (i, 0)),
        _full(wa.shape),
        _full(ba.shape),
    ]
    args = [S, xs, dinv, wa, ba]
    if wb is not None:
        in_specs.append(_full(wb.shape))
        args.append(wb)
    return pl.pallas_call(
        functools.partial(_tc_mid_body, pre_bias=pre_bias),
        grid=grid,
        in_specs=in_specs,
        out_specs=pl.BlockSpec((B, fout), lambda i: (i, 0)),
        out_shape=jax.ShapeDtypeStruct((N, fout), jnp.float32),
    )(*args)


def _tc4(S, xs, dinv, b3r, batch2, fW1, fb1r, fW2, fb2r, fW3, fb3r):
    args = [S, xs, dinv, b3r, batch2, fW1, fb1r, fW2, fb2r, fW3, fb3r]
    in_specs = [pl.BlockSpec((N, F), lambda i: (0, 0))]
    in_specs += [pl.BlockSpec(a.shape, lambda i, _r=len(a.shape): (0,) * _r)
                 for a in args[1:]]
    return pl.pallas_call(
        _tc4_body,
        grid=(1,),
        in_specs=in_specs,
        out_specs=pl.BlockSpec((G, fW3.shape[1]), lambda i: (0, 0)),
        out_shape=jax.ShapeDtypeStruct((G, fW3.shape[1]), jnp.float32),
    )(*args)


def kernel(x, edge_index, batch, W1, b1, W2, b2, W3, b3,
           fW1, fb1, fW2, fb2, fW3, fb3):
    x = x.astype(jnp.float32)
    src = edge_index[0]
    dst = edge_index[1]
    zrow = jnp.zeros((ZR, F), jnp.float32)
    batch2 = batch.reshape(N, 1)
    scat = _make_sc_scatter()

    degp = _make_sc_degree()(dst)
    dinv, xs1 = _tc1(degp, x)

    S1 = scat(xs1, src, dst, zrow)
    xs2 = _tc_mid(S1, xs1, dinv, W1, b1.reshape(1, -1), W2,
                  pre_bias=False, fout=128)
    S2 = scat(xs2, src, dst, zrow)
    xs3 = _tc_mid(S2, xs2, dinv, W3, b2.reshape(1, -1), None,
                  pre_bias=True, fout=128)
    S3 = scat(xs3, src, dst, zrow)
    out = _tc4(S3, xs3, dinv, b3.reshape(1, -1), batch2,
               fW1, fb1.reshape(1, -1), fW2, fb2.reshape(1, -1),
               fW3, fb3.reshape(1, -1))
    return out


# speedup vs baseline: 13.6519x; 1.2869x over previous
import functools

import jax
import jax.numpy as jnp
from jax import lax
from jax.experimental import pallas as pl
from jax.experimental.pallas import tpu as pltpu
from jax.experimental.pallas import tpu_sc as plsc

N = 10000
E = 320000
G = 64
F = 128

NC = 2
NS = 16
NW = NC * NS
EPT = E // NW
EPS = E // NS
K = 80
NCH = EPS // K
HALF = 5120
PAD = 128
ACR = HALF + PAD
ZR = ACR // NS
WR = HALF // NS
AR = NC * HALF


@functools.cache
def _sc_mesh():
    return plsc.VectorSubcoreMesh(core_axis_name="c", subcore_axis_name="s",
                                  num_cores=NC, num_subcores=NS)


@functools.cache
def _make_sc_degree():
    @functools.partial(
        pl.kernel,
        out_type=jax.ShapeDtypeStruct((NW, 1, N), jnp.float32),
        mesh=_sc_mesh(),
        compiler_params=pltpu.CompilerParams(needs_layout_passes=False),
        scratch_types=[
            pltpu.VMEM((EPT,), jnp.int32),
            pltpu.VMEM((N,), jnp.float32),
        ],
    )
    def deg_kernel(dst_hbm, degp_hbm, dstv, deg):
        c = lax.axis_index("c")
        s = lax.axis_index("s")
        wid = s * NC + c

        def zero_body(i, _):
            deg[pl.ds(i * 16, 16)] = jnp.zeros((16,), jnp.float32)
            return 0

        lax.fori_loop(0, N // 16, zero_body, 0)
        pltpu.sync_copy(dst_hbm.at[pl.ds(wid * EPT, EPT)], dstv)
        ones = jnp.full((16,), 1.0, jnp.float32)

        def hist_body(i, _):
            iv = dstv[pl.ds(i * 16, 16)]
            plsc.addupdate_scatter(deg, [iv], ones)
            return 0

        lax.fori_loop(0, EPT // 16, hist_body, 0)
        pltpu.sync_copy(deg, degp_hbm.at[wid, 0])

    return deg_kernel


@functools.cache
def _make_sc_scatter():
    @functools.partial(
        pl.kernel,
        out_type=jax.ShapeDtypeStruct((AR, F), jnp.float32),
        mesh=_sc_mesh(),
        scratch_types=[
            pltpu.VMEM((EPS,), jnp.int32),
            pltpu.VMEM((2, 1, K), jnp.int32),
            pltpu.VMEM((2, K, F), jnp.float32),
            pltpu.VMEM((ZR, F), jnp.float32),
            pltpu.VMEM_SHARED((ACR, F), jnp.float32),
            pltpu.SemaphoreType.DMA,
            pltpu.SemaphoreType.DMA,
        ],
    )
    def scat(xs_hbm, src_hbm, dst_hbm, zrow_hbm, out_hbm,
             sidx, dstv, rows, bounce, acc, g0, g1):
        c = lax.axis_index("c")
        s = lax.axis_index("s")
        base = s * EPS
        row0 = c * HALF
        dump = HALF + s * 8
        gsem = (g0, g1)

        pltpu.sync_copy(zrow_hbm, bounce)
        pltpu.sync_copy(src_hbm.at[pl.ds(base, EPS)], sidx)
        pltpu.sync_copy(bounce, acc.at[pl.ds(s * ZR, ZR)])
        plsc.subcore_barrier()

        def load_and_gather(i, b):
            pltpu.async_copy(xs_hbm.at[sidx.at[pl.ds(i * K, K)]],
                             rows.at[b], gsem[b])

        for b in range(2):
            load_and_gather(b, b)

        def consume(i, b):
            pltpu.sync_copy(dst_hbm.at[pl.ds(base + i * K, K)], dstv.at[b, 0])
            for j in range(K // 16):
                dv = dstv[b, 0, pl.ds(j * 16, 16)] - row0
                ok = (dv >= 0) & (dv < HALF)
                dstv[b, 0, pl.ds(j * 16, 16)] = jnp.where(ok, dv, dump)
            pltpu.make_async_copy(xs_hbm.at[sidx.at[pl.ds(i * K, K)]],
                                  rows.at[b], gsem[b]).wait()
            pltpu.sync_copy(rows.at[b], acc.at[dstv.at[b, 0]], add=True)

        def pair_body(t, _):
            for b in range(2):
                i = 2 * t + b
                consume(i, b)

                @pl.when(i + 2 < NCH)
                def _():
                    load_and_gather(i + 2, b)
            return 0

        lax.fori_loop(0, NCH // 2, pair_body, 0)
        if NCH % 2:
            consume(NCH - 1, 0)

        plsc.subcore_barrier()
        pltpu.sync_copy(acc.at[pl.ds(s * WR, WR)], bounce.at[pl.ds(0, WR)])
        pltpu.sync_copy(bounce.at[pl.ds(0, WR)],
                        out_hbm.at[pl.ds(row0 + s * WR, WR)])

    return scat


def _tc1_body(degp_ref, x_ref, dinv_ref, xs1_ref):
    ones = jnp.ones((NW, 1), jnp.float32)
    dp = degp_ref[...][:, 0, :]
    deg = lax.dot_general(dp, ones, (((0,), (0,)), ((), ())),
                          preferred_element_type=jnp.float32) + 1.0
    dinv = lax.rsqrt(deg)
    dinv_ref[...] = dinv
    xs1_ref[...] = x_ref[...] * dinv


def _tc_mid_body(s_ref, xs_ref, dinv_ref, wa_ref, ba_ref, *rest, pre_bias):
    dinv = dinv_ref[...]
    agg = (s_ref[...] + xs_ref[...]) * dinv
    if pre_bias:
        (out_ref,) = rest
        h = jnp.maximum(agg + ba_ref[...], 0.0)
        h = lax.dot_general(h, wa_ref[...], (((1,), (0,)), ((), ())),
                            preferred_element_type=jnp.float32)
        h = jnp.concatenate([h, jnp.zeros_like(h)], axis=1)
    else:
        wb_ref, out_ref = rest
        h = lax.dot_general(agg, wa_ref[...], (((1,), (0,)), ((), ())),
                            preferred_element_type=jnp.float32)
        h = jnp.maximum(h + ba_ref[...], 0.0)
        h = lax.dot_general(h, wb_ref[...], (((1,), (0,)), ((), ())),
                            preferred_element_type=jnp.float32)
    out_ref[...] = h * dinv


def _tc4_body(s_ref, xs_ref, dinv_ref, b3_ref, batch_ref,
              fw1_ref, fb1_ref, fw2_ref, fb2_ref, fw3_ref, fb3_ref, out_ref):
    dinv = dinv_ref[...]
    h3 = jnp.maximum((s_ref[...][:, :64] + xs_ref[...][:, :64]) * dinv
                     + b3_ref[...], 0.0)
    batch = batch_ref[...]

    gids = lax.broadcasted_iota(jnp.int32, (G, 1), 0)

    def seg_body(g, pool):
        mask = batch == g
        mg = jnp.max(jnp.where(mask, h3, -jnp.inf), axis=0)
        return jnp.where(gids == g, mg[None, :], pool)

    pool = lax.fori_loop(0, G, seg_body,
                         jnp.full((G, h3.shape[1]), -jnp.inf, jnp.float32))

    def mm(a, w_ref, b_ref):
        return lax.dot_general(a, w_ref[...], (((1,), (0,)), ((), ())),
                               preferred_element_type=jnp.float32) + b_ref[...]

    z = jnp.maximum(mm(pool, fw1_ref, fb1_ref), 0.0)
    z = jnp.maximum(mm(z, fw2_ref, fb2_ref), 0.0)
    z = mm(z, fw3_ref, fb3_ref)
    m = jnp.max(z, axis=1, keepdims=True)
    lse = m + jnp.log(jnp.sum(jnp.exp(z - m), axis=1, keepdims=True))
    out_ref[...] = z - lse


def _full(shape):
    return pl.BlockSpec(shape, lambda *_: tuple(0 for _ in shape))


def _tc1(degp, x):
    return pl.pallas_call(
        _tc1_body,
        out_shape=[jax.ShapeDtypeStruct((N, 1), jnp.float32),
                   jax.ShapeDtypeStruct((N, x.shape[1]), jnp.float32)],
    )(degp, x)


def _tc_mid(S, xs, dinv, wa, ba, wb, *, pre_bias, fout):
    B = 2000
    Fin = xs.shape[1]
    grid = (N // B,)
    in_specs = [
        pl.BlockSpec((B, F), lambda i: (i, 0)),
        pl.BlockSpec((B, Fin), lambda i: (i, 0)),
        pl.BlockSpec((B, 1), lambda i: (i, 0)),
        _full(wa.shape),
        _full(ba.shape),
    ]
    args = [S, xs, dinv, wa, ba]
    if wb is not None:
        in_specs.append(_full(wb.shape))
        args.append(wb)
    return pl.pallas_call(
        functools.partial(_tc_mid_body, pre_bias=pre_bias),
        grid=grid,
        in_specs=in_specs,
        out_specs=pl.BlockSpec((B, fout), lambda i: (i, 0)),
        out_shape=jax.ShapeDtypeStruct((N, fout), jnp.float32),
    )(*args)


def _tc4(S, xs, dinv, b3r, batch2, fW1, fb1r, fW2, fb2r, fW3, fb3r):
    args = [S, xs, dinv, b3r, batch2, fW1, fb1r, fW2, fb2r, fW3, fb3r]
    in_specs = [pl.BlockSpec((N, F), lambda i: (0, 0))]
    in_specs += [pl.BlockSpec(a.shape, lambda i, _r=len(a.shape): (0,) * _r)
                 for a in args[1:]]
    return pl.pallas_call(
        _tc4_body,
        grid=(1,),
        in_specs=in_specs,
        out_specs=pl.BlockSpec((G, fW3.shape[1]), lambda i: (0, 0)),
        out_shape=jax.ShapeDtypeStruct((G, fW3.shape[1]), jnp.float32),
    )(*args)


def kernel(x, edge_index, batch, W1, b1, W2, b2, W3, b3,
           fW1, fb1, fW2, fb2, fW3, fb3):
    x = x.astype(jnp.float32)
    src = edge_index[0]
    dst = edge_index[1]
    zrow = jnp.zeros((ZR, F), jnp.float32)
    batch2 = batch.reshape(N, 1)
    scat = _make_sc_scatter()

    degp = _make_sc_degree()(dst)
    dinv, xs1 = _tc1(degp, x)

    S1 = scat(xs1, src, dst, zrow)
    xs2 = _tc_mid(S1, xs1, dinv, W1, b1.reshape(1, -1), W2,
                  pre_bias=False, fout=128)
    S2 = scat(xs2, src, dst, zrow)
    xs3 = _tc_mid(S2, xs2, dinv, W3, b2.reshape(1, -1), None,
                  pre_bias=True, fout=128)
    S3 = scat(xs3, src, dst, zrow)
    out = _tc4(S3, xs3, dinv, b3.reshape(1, -1), batch2,
               fW1, fb1.reshape(1, -1), fW2, fb2.reshape(1, -1),
               fW3, fb3.reshape(1, -1))
    return out


# speedup vs baseline: 14.5836x; 1.0682x over previous
import functools

import jax
import jax.numpy as jnp
from jax import lax
from jax.experimental import pallas as pl
from jax.experimental.pallas import tpu as pltpu
from jax.experimental.pallas import tpu_sc as plsc

N = 10000
E = 320000
G = 64
F = 128

NC = 2
NS = 16
NW = NC * NS
EPT = E // NW
EPS = E // NS
K = 96
EPS2 = ((EPS + K - 1) // K) * K
NCH = EPS2 // K
LT = NS * EPS2
NB = 3
HALF = 5120
PAD = 128
ACR = HALF + PAD
ZR = ACR // NS
WR = HALF // NS
AR = NC * HALF


@functools.cache
def _sc_mesh():
    return plsc.VectorSubcoreMesh(core_axis_name="c", subcore_axis_name="s",
                                  num_cores=NC, num_subcores=NS)


@functools.cache
def _make_sc_degree():
    @functools.partial(
        pl.kernel,
        out_type=jax.ShapeDtypeStruct((NW, 1, N), jnp.float32),
        mesh=_sc_mesh(),
        compiler_params=pltpu.CompilerParams(needs_layout_passes=False),
        scratch_types=[
            pltpu.VMEM((EPT,), jnp.int32),
            pltpu.VMEM((N,), jnp.float32),
        ],
    )
    def deg_kernel(dst_hbm, degp_hbm, dstv, deg):
        c = lax.axis_index("c")
        s = lax.axis_index("s")
        wid = s * NC + c

        def zero_body(i, _):
            deg[pl.ds(i * 16, 16)] = jnp.zeros((16,), jnp.float32)
            return 0

        lax.fori_loop(0, N // 16, zero_body, 0)
        pltpu.sync_copy(dst_hbm.at[pl.ds(wid * EPT, EPT)], dstv)
        ones = jnp.full((16,), 1.0, jnp.float32)

        def hist_body(i, _):
            iv = dstv[pl.ds(i * 16, 16)]
            plsc.addupdate_scatter(deg, [iv], ones)
            return 0

        lax.fori_loop(0, EPT // 16, hist_body, 0)
        pltpu.sync_copy(deg, degp_hbm.at[wid, 0])

    return deg_kernel


@functools.cache
def _make_sc_scatter():
    @functools.partial(
        pl.kernel,
        out_type=jax.ShapeDtypeStruct((AR, F), jnp.float32),
        mesh=_sc_mesh(),
        scratch_types=[
            pltpu.VMEM((EPS2,), jnp.int32),
            pltpu.VMEM((NB, 1, K), jnp.int32),
            pltpu.VMEM((NB, K, F), jnp.float32),
            pltpu.VMEM_SHARED((ACR, F), jnp.float32),
            pltpu.SemaphoreType.DMA,
            pltpu.SemaphoreType.DMA,
            pltpu.SemaphoreType.DMA,
            pltpu.SemaphoreType.DMA,
            pltpu.SemaphoreType.DMA,
            pltpu.SemaphoreType.DMA,
        ],
    )
    def scat(xs_hbm, src_hbm, dst2_hbm, zrow_hbm, out_hbm,
             sidx, dstv, rows, acc, g0, g1, g2, s0, s1, s2):
        c = lax.axis_index("c")
        s = lax.axis_index("s")
        base = s * EPS2
        row0 = c * HALF
        gsem = (g0, g1, g2)
        ssem = (s0, s1, s2)

        pltpu.sync_copy(src_hbm.at[pl.ds(base, EPS2)], sidx)
        pltpu.sync_copy(zrow_hbm, acc.at[pl.ds(s * ZR, ZR)])
        plsc.subcore_barrier()

        def gather(i, b):
            pltpu.async_copy(xs_hbm.at[sidx.at[pl.ds(i * K, K)]],
                             rows.at[b], gsem[b])

        def chunk(i, b):
            nb = (b + 2) % NB
            pltpu.sync_copy(dst2_hbm.at[pl.ds(c * LT + base + i * K, K)],
                            dstv.at[b, 0])
            pltpu.make_async_copy(xs_hbm.at[sidx.at[pl.ds(i * K, K)]],
                                  rows.at[b], gsem[b]).wait()

            @pl.when(i >= 1)
            def _():
                pltpu.make_async_copy(rows.at[nb], acc.at[dstv.at[nb, 0]],
                                      ssem[nb]).wait()

            @pl.when(i + 2 < NCH)
            def _():
                gather(i + 2, nb)

            pltpu.async_copy(rows.at[b], acc.at[dstv.at[b, 0]], ssem[b],
                             add=True)

        gather(0, 0)
        gather(1, 1)

        def trip_body(t, _):
            for b in range(NB):
                chunk(NB * t + b, b)
            return 0

        lax.fori_loop(0, NCH // NB, trip_body, 0)
        for r in range(NCH % NB):
            chunk(NCH - (NCH % NB) + r, r)
        fb = (NCH - 1) % NB
        pltpu.make_async_copy(rows.at[fb], acc.at[dstv.at[fb, 0]],
                              ssem[fb]).wait()

        plsc.subcore_barrier()
        pltpu.sync_copy(acc.at[pl.ds(s * WR, WR)],
                        out_hbm.at[pl.ds(row0 + s * WR, WR)])

    return scat


def _tc1_body(degp_ref, x_ref, dstp_ref, dinv_ref, xs1_ref, dst2_ref):
    ones = jnp.ones((NW, 1), jnp.float32)
    dp = degp_ref[...][:, 0, :]
    deg = lax.dot_general(dp, ones, (((0,), (0,)), ((), ())),
                          preferred_element_type=jnp.float32) + 1.0
    dinv = lax.rsqrt(deg)
    dinv_ref[...] = dinv
    xs1_ref[...] = x_ref[...] * dinv
    dstp = dstp_ref[...]
    dumpv = HALF + (lax.broadcasted_iota(jnp.int32, (1, LT), 1) & (PAD - 1))
    for cc in range(NC):
        dl = dstp - cc * HALF
        ok = (dl >= 0) & (dl < HALF)
        dst2_ref[cc] = jnp.where(ok, dl, dumpv)


def _tc_mid_body(s_ref, xs_ref, dinv_ref, wa_ref, ba_ref, *rest, pre_bias):
    dinv = dinv_ref[...]
    agg = (s_ref[...] + xs_ref[...]) * dinv
    if pre_bias:
        (out_ref,) = rest
        h = jnp.maximum(agg + ba_ref[...], 0.0)
        h = lax.dot_general(h, wa_ref[...], (((1,), (0,)), ((), ())),
                            preferred_element_type=jnp.float32)
        h = jnp.concatenate([h, jnp.zeros_like(h)], axis=1)
    else:
        wb_ref, out_ref = rest
        h = lax.dot_general(agg, wa_ref[...], (((1,), (0,)), ((), ())),
                            preferred_element_type=jnp.float32)
        h = jnp.maximum(h + ba_ref[...], 0.0)
        h = lax.dot_general(h, wb_ref[...], (((1,), (0,)), ((), ())),
                            preferred_element_type=jnp.float32)
    out_ref[...] = h * dinv


def _tc4_body(s_ref, xs_ref, dinv_ref, b3_ref, batch_ref,
              fw1_ref, fb1_ref, fw2_ref, fb2_ref, fw3_ref, fb3_ref, out_ref):
    dinv = dinv_ref[...]
    h3 = jnp.maximum((s_ref[...][:, :64] + xs_ref[...][:, :64]) * dinv
                     + b3_ref[...], 0.0)
    batch = batch_ref[...]

    gids = lax.broadcasted_iota(jnp.int32, (G, 1), 0)

    def seg_body(g, pool):
        mask = batch == g
        mg = jnp.max(jnp.where(mask, h3, -jnp.inf), axis=0)
        return jnp.where(gids == g, mg[None, :], pool)

    pool = lax.fori_loop(0, G, seg_body,
                         jnp.full((G, h3.shape[1]), -jnp.inf, jnp.float32))

    def mm(a, w_ref, b_ref):
        return lax.dot_general(a, w_ref[...], (((1,), (0,)), ((), ())),
                               preferred_element_type=jnp.float32) + b_ref[...]

    z = jnp.maximum(mm(pool, fw1_ref, fb1_ref), 0.0)
    z = jnp.maximum(mm(z, fw2_ref, fb2_ref), 0.0)
    z = mm(z, fw3_ref, fb3_ref)
    m = jnp.max(z, axis=1, keepdims=True)
    lse = m + jnp.log(jnp.sum(jnp.exp(z - m), axis=1, keepdims=True))
    out_ref[...] = z - lse


def _full(shape):
    return pl.BlockSpec(shape, lambda *_: tuple(0 for _ in shape))


def _tc1(degp, x, dstp):
    return pl.pallas_call(
        _tc1_body,
        out_shape=[jax.ShapeDtypeStruct((N, 1), jnp.float32),
                   jax.ShapeDtypeStruct((N, x.shape[1]), jnp.float32),
                   jax.ShapeDtypeStruct((NC, 1, LT), jnp.int32)],
    )(degp, x, dstp)


def _tc_mid(S, xs, dinv, wa, ba, wb, *, pre_bias, fout):
    B = 2000
    Fin = xs.shape[1]
    grid = (N // B,)
    in_specs = [
        pl.BlockSpec((B, F), lambda i: (i, 0)),
        pl.BlockSpec((B, Fin), lambda i: (i, 0)),
        pl.BlockSpec((B, 1), lambda i: (i, 0)),
        _full(wa.shape),
        _full(ba.shape),
    ]
    args = [S, xs, dinv, wa, ba]
    if wb is not None:
        in_specs.append(_full(wb.shape))
        args.append(wb)
    return pl.pallas_call(
        functools.partial(_tc_mid_body, pre_bias=pre_bias),
        grid=grid,
        in_specs=in_specs,
        out_specs=pl.BlockSpec((B, fout), lambda i: (i, 0)),
        out_shape=jax.ShapeDtypeStruct((N, fout), jnp.float32),
    )(*args)


def _tc4(S, xs, dinv, b3r, batch2, fW1, fb1r, fW2, fb2r, fW3, fb3r):
    args = [S, xs, dinv, b3r, batch2, fW1, fb1r, fW2, fb2r, fW3, fb3r]
    in_specs = [pl.BlockSpec((N, F), lambda i: (0, 0))]
    in_specs += [pl.BlockSpec(a.shape, lambda i, _r=len(a.shape): (0,) * _r)
                 for a in args[1:]]
    return pl.pallas_call(
        _tc4_body,
        grid=(1,),
        in_specs=in_specs,
        out_specs=pl.BlockSpec((G, fW3.shape[1]), lambda i: (0, 0)),
        out_shape=jax.ShapeDtypeStruct((G, fW3.shape[1]), jnp.float32),
    )(*args)


def kernel(x, edge_index, batch, W1, b1, W2, b2, W3, b3,
           fW1, fb1, fW2, fb2, fW3, fb3):
    x = x.astype(jnp.float32)
    src = edge_index[0]
    dst = edge_index[1]
    zrow = jnp.zeros((ZR, F), jnp.float32)
    batch2 = batch.reshape(N, 1)
    pad = jnp.zeros((NS, EPS2 - EPS), jnp.int32)
    srcp = jnp.concatenate([src.reshape(NS, EPS), pad], axis=1).reshape(-1)
    dstp = jnp.concatenate([dst.reshape(NS, EPS), pad + (1 << 20)],
                           axis=1).reshape(1, -1)
    scat = _make_sc_scatter()

    degp = _make_sc_degree()(dst)
    dinv, xs1, dst2 = _tc1(degp, x, dstp)

    dst2 = dst2.reshape(-1)
    S1 = scat(xs1, srcp, dst2, zrow)
    xs2 = _tc_mid(S1, xs1, dinv, W1, b1.reshape(1, -1), W2,
                  pre_bias=False, fout=128)
    S2 = scat(xs2, srcp, dst2, zrow)
    xs3 = _tc_mid(S2, xs2, dinv, W3, b2.reshape(1, -1), None,
                  pre_bias=True, fout=128)
    S3 = scat(xs3, srcp, dst2, zrow)
    out = _tc4(S3, xs3, dinv, b3.reshape(1, -1), batch2,
               fW1, fb1.reshape(1, -1), fW2, fb2.reshape(1, -1),
               fW3, fb3.reshape(1, -1))
    return out
